# Initial kernel scaffold; baseline (speedup 1.0000x reference)
import jax
import jax.numpy as jnp
from jax.experimental import pallas as pl


def kernel(feat, speaker_feat, spatial_feat, index, W1, b1, W2, b2, We, be):
    raise NotImplementedError("write your pallas kernel here")


# speedup vs baseline: 17.7371x; 17.7371x over previous
import functools

import jax
import jax.numpy as jnp
from jax import lax
from jax.experimental import pallas as pl
from jax.experimental.pallas import tpu as pltpu
from jax.experimental.pallas import tpu_sc as plsc

N = 10000
E = 320000
N_PAD = 10240
NC = 2
NS = 16
L = 16
NW = NC * NS
EP = E // NW
NT = N_PAD // NS
BLK = 512
GRID = N_PAD // BLK

_NEG = float("-inf")


def _tc1_body(gf_ref, gs_ref, gp_ref, gb_ref, feat_ref, spk_ref, spa_ref,
              y_ref):
    dn = (((1,), (1,)), ((), ()))
    acc = lax.dot_general(gf_ref[...], feat_ref[...], dn,
                          preferred_element_type=jnp.float32,
                          precision=lax.Precision.HIGHEST)
    acc += lax.dot_general(gs_ref[...], spk_ref[...], dn,
                           preferred_element_type=jnp.float32,
                           precision=lax.Precision.HIGHEST)
    acc += lax.dot_general(gp_ref[...], spa_ref[...], dn,
                           preferred_element_type=jnp.float32,
                           precision=lax.Precision.HIGHEST)
    y_ref[...] = acc + gb_ref[...]


def _tc1(gf, gs, gp, gb, feat_p, spk_p, spa_p):
    return pl.pallas_call(
        _tc1_body,
        grid=(GRID,),
        in_specs=[
            pl.BlockSpec((4, 128), lambda i: (0, 0)),
            pl.BlockSpec((4, 16), lambda i: (0, 0)),
            pl.BlockSpec((4, 16), lambda i: (0, 0)),
            pl.BlockSpec((4, 1), lambda i: (0, 0)),
            pl.BlockSpec((BLK, 128), lambda i: (i, 0)),
            pl.BlockSpec((BLK, 16), lambda i: (i, 0)),
            pl.BlockSpec((BLK, 16), lambda i: (i, 0)),
        ],
        out_specs=pl.BlockSpec((4, BLK), lambda i: (0, i)),
        out_shape=jax.ShapeDtypeStruct((4, N_PAD), jnp.float32),
    )(gf, gs, gp, gb, feat_p, spk_p, spa_p)


def _scatter_max(mref, idx, val):
    cur = plsc.load_gather(mref, [idx])

    def cond(needs):
        return jnp.any(needs)

    def body(needs):
        plsc.store_scatter(mref, [idx], val, mask=needs)
        cur2 = plsc.load_gather(mref, [idx])
        return jnp.logical_and(needs, cur2 < val)

    lax.while_loop(cond, body, cur < val)


def _sc_body(y_hbm, src_hbm, dst_hbm, part_hbm, b1_v, b2_v, m1_v, m2_v,
             src_v, dst_v, buf_v, out_v, shared):
    c = lax.axis_index("c")
    s = lax.axis_index("s")
    g = c * NS + s

    pltpu.sync_copy(y_hbm.at[0], b1_v)
    pltpu.sync_copy(y_hbm.at[1], b2_v)
    pltpu.sync_copy(src_hbm.at[pl.ds(g * EP, EP)], src_v)
    pltpu.sync_copy(dst_hbm.at[pl.ds(g * EP, EP)], dst_v)

    def init(k, _):
        m1_v[pl.ds(k * L, L)] = jnp.full((L,), _NEG, jnp.float32)
        m2_v[pl.ds(k * L, L)] = jnp.full((L,), _NEG, jnp.float32)
        return 0

    lax.fori_loop(0, N_PAD // L, init, 0)

    def edge_group(e, _):
        s16 = src_v[pl.ds(e * L, L)]
        d16 = dst_v[pl.ds(e * L, L)]
        _scatter_max(m1_v, d16, plsc.load_gather(b1_v, [s16]))
        _scatter_max(m2_v, d16, plsc.load_gather(b2_v, [s16]))
        return 0

    lax.fori_loop(0, EP // L, edge_group, 0)

    pltpu.sync_copy(m1_v, shared.at[s, 0])
    pltpu.sync_copy(m2_v, shared.at[s, 1])
    plsc.subcore_barrier()

    n0 = s * NT
    for comp in range(2):
        for t in range(NS):
            pltpu.sync_copy(shared.at[t, comp, pl.ds(n0, NT)], buf_v.at[t])

        def merge(k, _):
            acc = buf_v[0, pl.ds(k * L, L)]
            for t in range(1, NS):
                acc = jnp.maximum(acc, buf_v[t, pl.ds(k * L, L)])
            out_v[pl.ds(k * L, L)] = acc
            return 0

        lax.fori_loop(0, NT // L, merge, 0)
        pltpu.sync_copy(out_v, part_hbm.at[c, comp, pl.ds(n0, NT)])


def _sc(y, src, dst):
    mesh = plsc.VectorSubcoreMesh(core_axis_name="c", subcore_axis_name="s",
                                  num_cores=NC, num_subcores=NS)
    return pl.kernel(
        _sc_body,
        out_type=jax.ShapeDtypeStruct((NC, 2, N_PAD), jnp.float32),
        mesh=mesh,
        compiler_params=pltpu.CompilerParams(needs_layout_passes=False),
        scratch_types=[
            pltpu.VMEM((N_PAD,), jnp.float32),
            pltpu.VMEM((N_PAD,), jnp.float32),
            pltpu.VMEM((N_PAD,), jnp.float32),
            pltpu.VMEM((N_PAD,), jnp.float32),
            pltpu.VMEM((EP,), jnp.int32),
            pltpu.VMEM((EP,), jnp.int32),
            pltpu.VMEM((NS, NT), jnp.float32),
            pltpu.VMEM((NT,), jnp.float32),
            pltpu.VMEM_SHARED((NS, 2, N_PAD), jnp.float32),
        ],
    )(y, src, dst)


def _tc2_body(p_ref, c_ref, o_ref):
    m = jnp.maximum(p_ref[0], p_ref[1])
    o_ref[...] = jnp.where(m == _NEG, 0.0, c_ref[...] + m)


def _tc2(part, cc):
    return pl.pallas_call(
        _tc2_body,
        out_shape=jax.ShapeDtypeStruct((2, N_PAD), jnp.float32),
    )(part, cc)


@jax.jit
def kernel(feat, speaker_feat, spatial_feat, index, W1, b1, W2, b2, We, be):
    Q = We[:, 64:]
    P = We[:, :64] - Q
    G = jnp.concatenate([Q, P], axis=0)
    gf = G @ (W1 + W2[:, :128])
    gs = G @ W2[:, 128:144]
    gp = G @ W2[:, 144:160]
    gb = G @ (b1 + b2) + jnp.concatenate([jnp.zeros((2,), jnp.float32), be])
    gb = gb[:, None]

    pad = ((0, N_PAD - N), (0, 0))
    feat_p = jnp.pad(feat, pad)
    spk_p = jnp.pad(speaker_feat, pad)
    spa_p = jnp.pad(spatial_feat, pad)

    y = _tc1(gf, gs, gp, gb, feat_p, spk_p, spa_p)
    part = _sc(y, index[0], index[1])
    out2 = _tc2(part, y[2:4])
    return out2[:, :N].T


# speedup vs baseline: 29.8814x; 1.6847x over previous
import jax
import jax.numpy as jnp
from jax import lax
from jax.experimental import pallas as pl
from jax.experimental.pallas import tpu as pltpu
from jax.experimental.pallas import tpu_sc as plsc

N = 10000
E = 320000
N_PAD = 10240
NC = 2
NS = 16
L = 16
NW = NC * NS
EP = E // NW
NT = N_PAD // NS

_NEG = float("-inf")
_DN = (((1,), (1,)), ((), ()))
_HI = lax.Precision.HIGHEST


def _tc1_body(feat_ref, spk_ref, spa_ref, w1_ref, w2_ref, we_ref, b12_ref,
              y_ref):
    cat = jnp.concatenate([feat_ref[...], spk_ref[...], spa_ref[...]], axis=1)
    x = lax.dot_general(feat_ref[...], w1_ref[...], _DN,
                        preferred_element_type=jnp.float32, precision=_HI)
    x += lax.dot_general(cat, w2_ref[...], _DN,
                         preferred_element_type=jnp.float32, precision=_HI)
    x += b12_ref[...]
    q = we_ref[:, 64:128]
    p = we_ref[:, 0:64] - q
    g = jnp.concatenate([q, p], axis=0)
    y_ref[...] = lax.dot_general(g, x, _DN,
                                 preferred_element_type=jnp.float32,
                                 precision=_HI)


def _tc1(feat, spk, spa, W1, W2, We, b12):
    return pl.pallas_call(
        _tc1_body,
        out_shape=jax.ShapeDtypeStruct((4, N), jnp.float32),
    )(feat, spk, spa, W1, W2, We, b12)


def _sc_body(y_hbm, idx_hbm, part_hbm, b1_v, b2_v, m1_v, m2_v,
             src_v, dst_v, buf_v, out_v, sem, shared):
    c = lax.axis_index("c")
    s = lax.axis_index("s")
    g = c * NS + s

    d1 = pltpu.async_copy(y_hbm.at[0], b1_v, sem)
    d2 = pltpu.async_copy(y_hbm.at[1], b2_v, sem)
    d3 = pltpu.async_copy(idx_hbm.at[pl.ds(g * EP, EP)], src_v, sem)
    d4 = pltpu.async_copy(idx_hbm.at[pl.ds(E + g * EP, EP)], dst_v, sem)

    neg = jnp.full((L,), _NEG, jnp.float32)

    def init(k, _):
        m1_v[pl.ds(k * L, L)] = neg
        m2_v[pl.ds(k * L, L)] = neg
        return 0

    lax.fori_loop(0, N_PAD // L, init, 0)
    d1.wait()
    d2.wait()
    d3.wait()
    d4.wait()

    def edge_group(e, _):
        s16 = src_v[pl.ds(e * L, L)]
        d16 = dst_v[pl.ds(e * L, L)]
        b1 = plsc.load_gather(b1_v, [s16])
        b2 = plsc.load_gather(b2_v, [s16])
        n1 = plsc.load_gather(m1_v, [d16]) < b1
        n2 = plsc.load_gather(m2_v, [d16]) < b2
        plsc.store_scatter(m1_v, [d16], b1, mask=n1)
        plsc.store_scatter(m2_v, [d16], b2, mask=n2)
        lost1 = jnp.logical_and(n1, plsc.load_gather(m1_v, [d16]) < b1)
        lost2 = jnp.logical_and(n2, plsc.load_gather(m2_v, [d16]) < b2)

        @pl.when(jnp.any(jnp.logical_or(lost1, lost2)))
        def _slow():
            def cond(carry):
                return jnp.any(jnp.logical_or(carry[0], carry[1]))

            def body(carry):
                l1, l2 = carry
                plsc.store_scatter(m1_v, [d16], b1, mask=l1)
                plsc.store_scatter(m2_v, [d16], b2, mask=l2)
                r1 = jnp.logical_and(l1, plsc.load_gather(m1_v, [d16]) < b1)
                r2 = jnp.logical_and(l2, plsc.load_gather(m2_v, [d16]) < b2)
                return (r1, r2)

            lax.while_loop(cond, body, (lost1, lost2))

        return 0

    lax.fori_loop(0, EP // L, edge_group, 0)

    pltpu.sync_copy(m1_v, shared.at[s, 0])
    pltpu.sync_copy(m2_v, shared.at[s, 1])
    plsc.subcore_barrier()

    n0 = s * NT
    descs = []
    for comp in range(2):
        for t in range(NS):
            descs.append(pltpu.async_copy(
                shared.at[t, comp, pl.ds(n0, NT)], buf_v.at[comp, t], sem))
    for d_ in descs:
        d_.wait()

    for comp in range(2):
        def merge(k, _, comp=comp):
            acc = buf_v[comp, 0, pl.ds(k * L, L)]
            for t in range(1, NS):
                acc = jnp.maximum(acc, buf_v[comp, t, pl.ds(k * L, L)])
            out_v[pl.ds(k * L, L)] = acc
            return 0

        lax.fori_loop(0, NT // L, merge, 0)
        pltpu.sync_copy(out_v, part_hbm.at[c, comp, pl.ds(n0, NT)])


def _sc(y, idx_flat):
    mesh = plsc.VectorSubcoreMesh(core_axis_name="c", subcore_axis_name="s",
                                  num_cores=NC, num_subcores=NS)
    return pl.kernel(
        _sc_body,
        out_type=jax.ShapeDtypeStruct((NC, 2, N_PAD), jnp.float32),
        mesh=mesh,
        compiler_params=pltpu.CompilerParams(needs_layout_passes=False),
        scratch_types=[
            pltpu.VMEM((N,), jnp.float32),
            pltpu.VMEM((N,), jnp.float32),
            pltpu.VMEM((N_PAD,), jnp.float32),
            pltpu.VMEM((N_PAD,), jnp.float32),
            pltpu.VMEM((EP,), jnp.int32),
            pltpu.VMEM((EP,), jnp.int32),
            pltpu.VMEM((2, NS, NT), jnp.float32),
            pltpu.VMEM((NT,), jnp.float32),
            pltpu.SemaphoreType.DMA,
            pltpu.VMEM_SHARED((NS, 2, N_PAD), jnp.float32),
        ],
    )(y, idx_flat)


def _tc2_body(p_ref, c_ref, be_ref, o_ref):
    m = jnp.maximum(p_ref[0, :, 0:N], p_ref[1, :, 0:N])
    o_ref[...] = jnp.where(m == _NEG, 0.0, c_ref[...] + be_ref[...] + m)


def _tc2(part, cc, be2):
    return pl.pallas_call(
        _tc2_body,
        out_shape=jax.ShapeDtypeStruct((2, N), jnp.float32),
    )(part, cc, be2)


@jax.jit
def kernel(feat, speaker_feat, spatial_feat, index, W1, b1, W2, b2, We, be):
    b12 = (b1 + b2)[None, :]
    be2 = be[:, None]
    idx_flat = index.reshape(-1)

    y = _tc1(feat, speaker_feat, spatial_feat, W1, W2, We, b12)
    part = _sc(y, idx_flat)
    out2 = _tc2(part, y[2:4], be2)
    return out2.T


# speedup vs baseline: 31.9838x; 1.0704x over previous
import jax
import jax.numpy as jnp
from jax import lax
from jax.experimental import pallas as pl
from jax.experimental.pallas import tpu as pltpu
from jax.experimental.pallas import tpu_sc as plsc

N = 10000
E = 320000
N_PAD = 10240
NC = 2
NS = 16
L = 16
NW = NC * NS
EP = E // NW
NT = N_PAD // NS
BLK = 1024
U = 5

_NEG = float("-inf")
_DN = (((1,), (1,)), ((), ()))
_HI = lax.Precision.HIGHEST


def _tc1_body(feat_ref, spk_ref, spa_ref, w1_ref, w2_ref, we_ref, b12_ref,
              y_ref):
    cat = jnp.concatenate([feat_ref[...], spk_ref[...], spa_ref[...]], axis=1)
    x = lax.dot_general(feat_ref[...], w1_ref[...], _DN,
                        preferred_element_type=jnp.float32, precision=_HI)
    x += lax.dot_general(cat, w2_ref[...], _DN,
                         preferred_element_type=jnp.float32, precision=_HI)
    x += b12_ref[...]
    q = we_ref[:, 64:128]
    p = we_ref[:, 0:64] - q
    g = jnp.concatenate([q, p], axis=0)
    y_ref[...] = lax.dot_general(g, x, _DN,
                                 preferred_element_type=jnp.float32,
                                 precision=_HI)


def _tc1(feat, spk, spa, W1, W2, We, b12):
    return pl.pallas_call(
        _tc1_body,
        grid=(N_PAD // BLK,),
        in_specs=[
            pl.BlockSpec((BLK, 128), lambda i: (i, 0)),
            pl.BlockSpec((BLK, 16), lambda i: (i, 0)),
            pl.BlockSpec((BLK, 16), lambda i: (i, 0)),
            pl.BlockSpec((64, 128), lambda i: (0, 0)),
            pl.BlockSpec((64, 160), lambda i: (0, 0)),
            pl.BlockSpec((2, 128), lambda i: (0, 0)),
            pl.BlockSpec((1, 64), lambda i: (0, 0)),
        ],
        out_specs=pl.BlockSpec((4, BLK), lambda i: (0, i)),
        out_shape=jax.ShapeDtypeStruct((4, N_PAD), jnp.float32),
    )(feat, spk, spa, W1, W2, We, b12)


def _take(v, idx):
    dn = lax.GatherDimensionNumbers(offset_dims=(), collapsed_slice_dims=(0,),
                                    start_index_map=(0,))
    return lax.gather(v, idx[:, None], dimension_numbers=dn, slice_sizes=(1,),
                      mode=lax.GatherScatterMode.PROMISE_IN_BOUNDS)


def _sc_body(y_hbm, idx_hbm, part_hbm, b1_v, b2_v, m1_v, m2_v,
             src_v, dst_v, buf_v, out_v, sem, shared):
    c = lax.axis_index("c")
    s = lax.axis_index("s")
    g = c * NS + s

    d1 = pltpu.async_copy(y_hbm.at[0], b1_v, sem)
    d2 = pltpu.async_copy(y_hbm.at[1], b2_v, sem)
    d3 = pltpu.async_copy(idx_hbm.at[0, pl.ds(g * EP, EP)], src_v, sem)
    d4 = pltpu.async_copy(idx_hbm.at[1, pl.ds(g * EP, EP)], dst_v, sem)

    neg = jnp.full((L,), _NEG, jnp.float32)

    def init(k, _):
        m1_v[pl.ds(k * L, L)] = neg
        m2_v[pl.ds(k * L, L)] = neg
        return 0

    lax.fori_loop(0, N_PAD // L, init, 0)
    d1.wait()
    d2.wait()
    d3.wait()
    d4.wait()

    iota = lax.iota(jnp.int32, L)
    isl15 = iota == (L - 1)
    offidx = [jnp.maximum(iota - (1 << k), 0) for k in range(4)]
    nxtidx = jnp.minimum(iota + 1, L - 1)

    def one_group(e):
        d16 = dst_v[pl.ds(e * L, L)]
        s16 = src_v[pl.ds(e * L, L)]
        dsort, perm = plsc.sort_key_val(d16, iota)
        s16p = _take(s16, perm)
        b1 = plsc.load_gather(b1_v, [s16p])
        b2 = plsc.load_gather(b2_v, [s16p])
        for k in range(4):
            keq = _take(dsort, offidx[k]) == dsort
            b1 = jnp.where(keq, jnp.maximum(b1, _take(b1, offidx[k])), b1)
            b2 = jnp.where(keq, jnp.maximum(b2, _take(b2, offidx[k])), b2)
        islast = jnp.logical_or(_take(dsort, nxtidx) != dsort, isl15)
        cur1 = plsc.load_gather(m1_v, [dsort])
        cur2 = plsc.load_gather(m2_v, [dsort])
        plsc.store_scatter(m1_v, [dsort], jnp.maximum(cur1, b1), mask=islast)
        plsc.store_scatter(m2_v, [dsort], jnp.maximum(cur2, b2), mask=islast)

    def edge_group(e, _):
        for u in range(U):
            one_group(e * U + u)
        return 0

    lax.fori_loop(0, EP // L // U, edge_group, 0)

    pltpu.sync_copy(m1_v, shared.at[s, 0])
    pltpu.sync_copy(m2_v, shared.at[s, 1])
    plsc.subcore_barrier()

    n0 = s * NT
    descs = []
    for comp in range(2):
        for t in range(NS):
            descs.append(pltpu.async_copy(
                shared.at[t, comp, pl.ds(n0, NT)], buf_v.at[comp, t], sem))
    for d_ in descs:
        d_.wait()

    for comp in range(2):
        def merge(k, _, comp=comp):
            acc = buf_v[comp, 0, pl.ds(k * L, L)]
            for t in range(1, NS):
                acc = jnp.maximum(acc, buf_v[comp, t, pl.ds(k * L, L)])
            out_v[pl.ds(k * L, L)] = acc
            return 0

        lax.fori_loop(0, NT // L, merge, 0)
        pltpu.sync_copy(out_v, part_hbm.at[c, comp, pl.ds(n0, NT)])


def _sc(y, index):
    mesh = plsc.VectorSubcoreMesh(core_axis_name="c", subcore_axis_name="s",
                                  num_cores=NC, num_subcores=NS)
    return pl.kernel(
        _sc_body,
        out_type=jax.ShapeDtypeStruct((NC, 2, N_PAD), jnp.float32),
        mesh=mesh,
        compiler_params=pltpu.CompilerParams(needs_layout_passes=False,
                                             use_tc_tiling_on_sc=False),
        scratch_types=[
            pltpu.VMEM((N_PAD,), jnp.float32),
            pltpu.VMEM((N_PAD,), jnp.float32),
            pltpu.VMEM((N_PAD,), jnp.float32),
            pltpu.VMEM((N_PAD,), jnp.float32),
            pltpu.VMEM((EP,), jnp.int32),
            pltpu.VMEM((EP,), jnp.int32),
            pltpu.VMEM((2, NS, NT), jnp.float32),
            pltpu.VMEM((NT,), jnp.float32),
            pltpu.SemaphoreType.DMA,
            pltpu.VMEM_SHARED((NS, 2, N_PAD), jnp.float32),
        ],
    )(y, index)


def _tc2_body(p_ref, c_ref, be_ref, o_ref):
    m = jnp.maximum(p_ref[0, :, 0:N], p_ref[1, :, 0:N])
    o_ref[...] = jnp.where(m == _NEG, 0.0, c_ref[...] + be_ref[...] + m)


def _tc2(part, cc, be2):
    return pl.pallas_call(
        _tc2_body,
        out_shape=jax.ShapeDtypeStruct((2, N), jnp.float32),
    )(part, cc, be2)


@jax.jit
def kernel(feat, speaker_feat, spatial_feat, index, W1, b1, W2, b2, We, be):
    b12 = (b1 + b2)[None, :]
    be2 = be[:, None]

    y = _tc1(feat, speaker_feat, spatial_feat, W1, W2, We, b12)
    part = _sc(y, index)
    out2 = _tc2(part, y[2:4, 0:N], be2)
    return out2.T


# speedup vs baseline: 37.3079x; 1.1665x over previous
import jax
import jax.numpy as jnp
from jax import lax
from jax.experimental import pallas as pl
from jax.experimental.pallas import tpu as pltpu
from jax.experimental.pallas import tpu_sc as plsc

N = 10000
E = 320000
N_PAD = 10240
NC = 2
NS = 16
L = 16
NW = NC * NS
EP = E // NW
NT = N_PAD // NS
BLK = 1024
U = 5

_NEG = float("-inf")
_DN = (((1,), (1,)), ((), ()))
_HI = lax.Precision.HIGHEST


def _tc1_body(feat_ref, spk_ref, spa_ref, wf_ref, ws_ref, wp_ref, we_ref,
              b12_ref, be_ref, y_ref):
    dnn = (((1,), (0,)), ((), ()))
    x = lax.dot_general(feat_ref[...], wf_ref[...], dnn,
                        preferred_element_type=jnp.float32)
    x += lax.dot_general(spk_ref[...], ws_ref[...], dnn,
                         preferred_element_type=jnp.float32)
    x += lax.dot_general(spa_ref[...], wp_ref[...], dnn,
                         preferred_element_type=jnp.float32)
    x += b12_ref[...]
    q = we_ref[:, 64:128]
    p = we_ref[:, 0:64] - q
    g = jnp.concatenate([q, p], axis=0)
    bias4 = jnp.concatenate([jnp.zeros((2, 1), jnp.float32), be_ref[...]],
                            axis=0)
    y_ref[...] = lax.dot_general(g, x, _DN,
                                 preferred_element_type=jnp.float32) + bias4


def _tc1(feat, spk, spa, Wf, Ws, Wp, We, b12, be2):
    return pl.pallas_call(
        _tc1_body,
        grid=(N_PAD // BLK,),
        in_specs=[
            pl.BlockSpec((BLK, 128), lambda i: (i, 0)),
            pl.BlockSpec((BLK, 16), lambda i: (i, 0)),
            pl.BlockSpec((BLK, 16), lambda i: (i, 0)),
            pl.BlockSpec((128, 64), lambda i: (0, 0)),
            pl.BlockSpec((16, 64), lambda i: (0, 0)),
            pl.BlockSpec((16, 64), lambda i: (0, 0)),
            pl.BlockSpec((2, 128), lambda i: (0, 0)),
            pl.BlockSpec((1, 64), lambda i: (0, 0)),
            pl.BlockSpec((2, 1), lambda i: (0, 0)),
        ],
        out_specs=pl.BlockSpec((4, BLK), lambda i: (0, i)),
        out_shape=jax.ShapeDtypeStruct((4, N_PAD), jnp.float32),
    )(feat, spk, spa, Wf, Ws, Wp, We, b12, be2)


def _take(v, idx):
    dn = lax.GatherDimensionNumbers(offset_dims=(), collapsed_slice_dims=(0,),
                                    start_index_map=(0,))
    return lax.gather(v, idx[:, None], dimension_numbers=dn, slice_sizes=(1,),
                      mode=lax.GatherScatterMode.PROMISE_IN_BOUNDS)


def _sc_body(y_hbm, idx_hbm, part_hbm, b1_v, b2_v, m1_v, m2_v,
             src_v, dst_v, buf_v, out_v, sem, shared):
    c = lax.axis_index("c")
    s = lax.axis_index("s")
    g = c * NS + s

    d1 = pltpu.async_copy(y_hbm.at[0], b1_v, sem)
    d2 = pltpu.async_copy(y_hbm.at[1], b2_v, sem)
    d3 = pltpu.async_copy(idx_hbm.at[0, pl.ds(g * EP, EP)], src_v, sem)
    d4 = pltpu.async_copy(idx_hbm.at[1, pl.ds(g * EP, EP)], dst_v, sem)

    neg = jnp.full((L,), _NEG, jnp.float32)

    def init(k, _):
        m1_v[pl.ds(k * L, L)] = neg
        m2_v[pl.ds(k * L, L)] = neg
        return 0

    with jax.named_scope("sc_stage"):
        lax.fori_loop(0, N_PAD // L, init, 0)
        d1.wait()
        d2.wait()
        d3.wait()
        d4.wait()

    iota = lax.iota(jnp.int32, L)
    isl15 = iota == (L - 1)
    offidx = [jnp.maximum(iota - (1 << k), 0) for k in range(4)]
    nxtidx = jnp.minimum(iota + 1, L - 1)

    def one_group(e):
        d16 = dst_v[pl.ds(e * L, L)]
        s16 = src_v[pl.ds(e * L, L)]
        dsort, perm = plsc.sort_key_val(d16, iota)
        s16p = _take(s16, perm)
        b1 = plsc.load_gather(b1_v, [s16p])
        b2 = plsc.load_gather(b2_v, [s16p])
        for k in range(4):
            keq = _take(dsort, offidx[k]) == dsort
            b1 = jnp.where(keq, jnp.maximum(b1, _take(b1, offidx[k])), b1)
            b2 = jnp.where(keq, jnp.maximum(b2, _take(b2, offidx[k])), b2)
        islast = jnp.logical_or(_take(dsort, nxtidx) != dsort, isl15)
        cur1 = plsc.load_gather(m1_v, [dsort])
        cur2 = plsc.load_gather(m2_v, [dsort])
        plsc.store_scatter(m1_v, [dsort], jnp.maximum(cur1, b1), mask=islast)
        plsc.store_scatter(m2_v, [dsort], jnp.maximum(cur2, b2), mask=islast)

    def edge_group(e, _):
        for u in range(U):
            one_group(e * U + u)
        return 0

    with jax.named_scope("sc_edges"):
        lax.fori_loop(0, EP // L // U, edge_group, 0)

    with jax.named_scope("sc_merge"):
        pltpu.sync_copy(m1_v, shared.at[s, 0])
        pltpu.sync_copy(m2_v, shared.at[s, 1])
        plsc.subcore_barrier()

        n0 = s * NT
        descs = []
        for comp in range(2):
            for t in range(NS):
                descs.append(pltpu.async_copy(
                    shared.at[t, comp, pl.ds(n0, NT)], buf_v.at[comp, t], sem))
        for d_ in descs:
            d_.wait()

        for comp in range(2):
            def merge(k, _, comp=comp):
                acc = buf_v[comp, 0, pl.ds(k * L, L)]
                for t in range(1, NS):
                    acc = jnp.maximum(acc, buf_v[comp, t, pl.ds(k * L, L)])
                out_v[pl.ds(k * L, L)] = acc
                return 0

            lax.fori_loop(0, NT // L, merge, 0)
            pltpu.sync_copy(out_v, part_hbm.at[c, comp, pl.ds(n0, NT)])


def _sc(y, index):
    mesh = plsc.VectorSubcoreMesh(core_axis_name="c", subcore_axis_name="s",
                                  num_cores=NC, num_subcores=NS)
    return pl.kernel(
        _sc_body,
        out_type=jax.ShapeDtypeStruct((NC, 2, N_PAD), jnp.float32),
        mesh=mesh,
        compiler_params=pltpu.CompilerParams(needs_layout_passes=False,
                                             use_tc_tiling_on_sc=False),
        scratch_types=[
            pltpu.VMEM((N_PAD,), jnp.float32),
            pltpu.VMEM((N_PAD,), jnp.float32),
            pltpu.VMEM((N_PAD,), jnp.float32),
            pltpu.VMEM((N_PAD,), jnp.float32),
            pltpu.VMEM((EP,), jnp.int32),
            pltpu.VMEM((EP,), jnp.int32),
            pltpu.VMEM((2, NS, NT), jnp.float32),
            pltpu.VMEM((NT,), jnp.float32),
            pltpu.SemaphoreType.DMA,
            pltpu.VMEM_SHARED((NS, 2, N_PAD), jnp.float32),
        ],
    )(y, index)


def _tc2_body(p_ref, c_ref, o_ref):
    m = jnp.maximum(p_ref[0, :, 0:N], p_ref[1, :, 0:N])
    o_ref[...] = jnp.where(m == _NEG, 0.0, c_ref[...] + m)


def _tc2(part, cc):
    return pl.pallas_call(
        _tc2_body,
        out_shape=jax.ShapeDtypeStruct((2, N), jnp.float32),
    )(part, cc)


@jax.jit
def kernel(feat, speaker_feat, spatial_feat, index, W1, b1, W2, b2, We, be):
    b12 = (b1 + b2)[None, :]
    be2 = be[:, None]
    Wf = (W1 + W2[:, :128]).T
    Ws = W2[:, 128:144].T
    Wp = W2[:, 144:160].T

    y = _tc1(feat, speaker_feat, spatial_feat, Wf, Ws, Wp, We, b12, be2)
    part = _sc(y, index)
    out2 = _tc2(part, y[2:4, 0:N])
    return out2.T


# speedup vs baseline: 40.9174x; 1.0967x over previous
import jax
import jax.numpy as jnp
from jax import lax
from jax.experimental import pallas as pl
from jax.experimental.pallas import tpu as pltpu
from jax.experimental.pallas import tpu_sc as plsc

N = 10000
E = 320000
N_PAD = 10240
NC = 2
NS = 16
L = 16
NW = NC * NS
EP = E // NW
NT = N_PAD // NS
BLK = 2048
U = 25

_NEG = float("-inf")
_DN = (((1,), (1,)), ((), ()))
_HI = lax.Precision.HIGHEST


def _tc1_body(feat_ref, spkt_ref, spat_ref, wf_ref, ws_ref, wp_ref, we_ref,
              b12_ref, be_ref, y_ref):
    dnn = (((1,), (0,)), ((), ()))
    dtt = (((0,), (0,)), ((), ()))
    x = lax.dot_general(feat_ref[...], wf_ref[...], dnn,
                        preferred_element_type=jnp.float32)
    x += lax.dot_general(spkt_ref[...], ws_ref[...], dtt,
                         preferred_element_type=jnp.float32)
    x += lax.dot_general(spat_ref[...], wp_ref[...], dtt,
                         preferred_element_type=jnp.float32)
    x += b12_ref[...]
    q = we_ref[:, 64:128]
    p = we_ref[:, 0:64] - q
    g = jnp.concatenate([q, p], axis=0)
    bias4 = jnp.concatenate([jnp.zeros((2, 1), jnp.float32), be_ref[...]],
                            axis=0)
    y_ref[...] = lax.dot_general(g, x, _DN,
                                 preferred_element_type=jnp.float32) + bias4


def _tc1(feat, spk, spa, Wf, Ws, Wp, We, b12, be2):
    return pl.pallas_call(
        _tc1_body,
        grid=(N_PAD // BLK,),
        in_specs=[
            pl.BlockSpec((BLK, 128), lambda i: (i, 0)),
            pl.BlockSpec((16, BLK), lambda i: (0, i)),
            pl.BlockSpec((16, BLK), lambda i: (0, i)),
            pl.BlockSpec((128, 64), lambda i: (0, 0)),
            pl.BlockSpec((16, 64), lambda i: (0, 0)),
            pl.BlockSpec((16, 64), lambda i: (0, 0)),
            pl.BlockSpec((2, 128), lambda i: (0, 0)),
            pl.BlockSpec((1, 64), lambda i: (0, 0)),
            pl.BlockSpec((2, 1), lambda i: (0, 0)),
        ],
        out_specs=pl.BlockSpec((4, BLK), lambda i: (0, i)),
        out_shape=jax.ShapeDtypeStruct((4, N_PAD), jnp.float32),
        compiler_params=pltpu.CompilerParams(
            dimension_semantics=("parallel",)),
    )(feat, spk, spa, Wf, Ws, Wp, We, b12, be2)


def _take(v, idx):
    dn = lax.GatherDimensionNumbers(offset_dims=(), collapsed_slice_dims=(0,),
                                    start_index_map=(0,))
    return lax.gather(v, idx[:, None], dimension_numbers=dn, slice_sizes=(1,),
                      mode=lax.GatherScatterMode.PROMISE_IN_BOUNDS)


def _sc_body(y_hbm, idx_hbm, part_hbm, b1_v, b2_v, m1_v, m2_v,
             src_v, dst_v, buf_v, out_v, sem, shared):
    c = lax.axis_index("c")
    s = lax.axis_index("s")
    g = c * NS + s

    d1 = pltpu.async_copy(y_hbm.at[0], b1_v, sem)
    d2 = pltpu.async_copy(y_hbm.at[1], b2_v, sem)
    d3 = pltpu.async_copy(idx_hbm.at[0, pl.ds(g * EP, EP)], src_v, sem)
    d4 = pltpu.async_copy(idx_hbm.at[1, pl.ds(g * EP, EP)], dst_v, sem)

    neg = jnp.full((L,), _NEG, jnp.float32)

    def init(k, _):
        m1_v[pl.ds(k * L, L)] = neg
        m2_v[pl.ds(k * L, L)] = neg
        return 0

    with jax.named_scope("sc_stage"):
        lax.fori_loop(0, N_PAD // L, init, 0)
        d1.wait()
        d2.wait()
        d3.wait()
        d4.wait()

    iota = lax.iota(jnp.int32, L)
    isl15 = iota == (L - 1)
    offidx = [jnp.maximum(iota - (1 << k), 0) for k in range(4)]
    nxtidx = jnp.minimum(iota + 1, L - 1)

    def one_group(e):
        d16 = dst_v[pl.ds(e * L, L)]
        s16 = src_v[pl.ds(e * L, L)]
        dsort, perm = plsc.sort_key_val(d16, iota)
        s16p = _take(s16, perm)
        b1 = plsc.load_gather(b1_v, [s16p])
        b2 = plsc.load_gather(b2_v, [s16p])
        for k in range(4):
            keq = _take(dsort, offidx[k]) == dsort
            b1 = jnp.where(keq, jnp.maximum(b1, _take(b1, offidx[k])), b1)
            b2 = jnp.where(keq, jnp.maximum(b2, _take(b2, offidx[k])), b2)
        islast = jnp.logical_or(_take(dsort, nxtidx) != dsort, isl15)
        cur1 = plsc.load_gather(m1_v, [dsort])
        cur2 = plsc.load_gather(m2_v, [dsort])
        plsc.store_scatter(m1_v, [dsort], jnp.maximum(cur1, b1), mask=islast)
        plsc.store_scatter(m2_v, [dsort], jnp.maximum(cur2, b2), mask=islast)

    def edge_group(e, _):
        for u in range(U):
            one_group(e * U + u)
        return 0

    with jax.named_scope("sc_edges"):
        lax.fori_loop(0, EP // L // U, edge_group, 0)

    with jax.named_scope("sc_merge"):
        pltpu.sync_copy(m1_v, shared.at[s, 0])
        pltpu.sync_copy(m2_v, shared.at[s, 1])
        plsc.subcore_barrier()

        n0 = s * NT
        descs = []
        for comp in range(2):
            for t in range(NS):
                descs.append(pltpu.async_copy(
                    shared.at[t, comp, pl.ds(n0, NT)], buf_v.at[comp, t], sem))
        for d_ in descs:
            d_.wait()

        for comp in range(2):
            def merge(k, _, comp=comp):
                acc = buf_v[comp, 0, pl.ds(k * L, L)]
                for t in range(1, NS):
                    acc = jnp.maximum(acc, buf_v[comp, t, pl.ds(k * L, L)])
                out_v[pl.ds(k * L, L)] = acc
                return 0

            lax.fori_loop(0, NT // L, merge, 0)
            pltpu.sync_copy(out_v, part_hbm.at[c, comp, pl.ds(n0, NT)])


def _sc(y, index):
    mesh = plsc.VectorSubcoreMesh(core_axis_name="c", subcore_axis_name="s",
                                  num_cores=NC, num_subcores=NS)
    return pl.kernel(
        _sc_body,
        out_type=jax.ShapeDtypeStruct((NC, 2, N_PAD), jnp.float32),
        mesh=mesh,
        compiler_params=pltpu.CompilerParams(needs_layout_passes=False,
                                             use_tc_tiling_on_sc=False),
        scratch_types=[
            pltpu.VMEM((N_PAD,), jnp.float32),
            pltpu.VMEM((N_PAD,), jnp.float32),
            pltpu.VMEM((N_PAD,), jnp.float32),
            pltpu.VMEM((N_PAD,), jnp.float32),
            pltpu.VMEM((EP,), jnp.int32),
            pltpu.VMEM((EP,), jnp.int32),
            pltpu.VMEM((2, NS, NT), jnp.float32),
            pltpu.VMEM((NT,), jnp.float32),
            pltpu.SemaphoreType.DMA,
            pltpu.VMEM_SHARED((NS, 2, N_PAD), jnp.float32),
        ],
    )(y, index)


def _tc2_body(p_ref, c_ref, o_ref):
    m = jnp.maximum(p_ref[0, :, 0:N], p_ref[1, :, 0:N])
    o_ref[...] = jnp.where(m == _NEG, 0.0, c_ref[...] + m)


def _tc2(part, cc):
    return pl.pallas_call(
        _tc2_body,
        out_shape=jax.ShapeDtypeStruct((2, N), jnp.float32),
    )(part, cc)


@jax.jit
def kernel(feat, speaker_feat, spatial_feat, index, W1, b1, W2, b2, We, be):
    b12 = (b1 + b2)[None, :]
    be2 = be[:, None]
    Wf = (W1 + W2[:, :128]).T
    Ws = W2[:, 128:144].T
    Wp = W2[:, 144:160].T

    y = _tc1(feat, speaker_feat.T, spatial_feat.T, Wf, Ws, Wp, We, b12, be2)
    part = _sc(y, index)
    out2 = _tc2(part, y[2:4, 0:N])
    return out2.T
